# baseline (speedup 1.0000x reference)
import functools

import jax
import jax.numpy as jnp
from jax import lax
from jax.experimental import pallas as pl
from jax.experimental.pallas import tpu as pltpu
from jax.experimental.pallas import tpu_sc as plsc

S = 7
C = 26
N_IMG = 16384
NW = 32
N_SLABS = N_IMG * S
SLABS_PER_W = N_SLABS // NW
SLABS_CH = 14
N_CHUNKS = SLABS_PER_W // SLABS_CH
CH_ROWS = SLABS_CH * S
N_GROUPS = (CH_ROWS + 15) // 16
RING = 4

LAMBDA_LOC = 10.0
LAMBDA_NOOBJ = 0.5


def _sqrt16(x):
    i = plsc.bitcast(x, jnp.int32)
    r = plsc.bitcast(jnp.int32(0x5F3759DF) - (i >> 1), jnp.float32)
    for _ in range(2):
        r = r * (1.5 - 0.5 * x * r * r)
    return x * r


def _corners(x, y, w, h):
    x64 = x * 64.0
    y64 = y * 64.0
    w224 = w * 224.0
    h224 = h * 224.0
    return x64 - w224, y64 - h224, x64 + w224, y64 + h224


def _area(b):
    return (b[2] - b[0] + 1.0) * (b[3] - b[1] + 1.0)


def _iou(b1, b2, a2):
    xA = jnp.maximum(b1[0], b2[0])
    yA = jnp.maximum(b1[1], b2[1])
    xB = jnp.minimum(b1[2], b2[2])
    yB = jnp.minimum(b1[3], b2[3])
    inter = jnp.maximum(0.0, xB - xA + 1.0) * jnp.maximum(0.0, yB - yA + 1.0)
    a1 = _area(b1)
    return inter / (a1 + a2 - inter)


def _group_loss(pbuf, tbuf, g, acc):
    lane = lax.iota(jnp.int32, 16)
    r_raw = lane + 16 * g
    r = jnp.minimum(r_raw, CH_ROWS - 1)
    r_f = r.astype(jnp.float32)
    slab_v = (r_f * (1.0 / S) + (0.5 / S)).astype(jnp.int32)
    b_v = r - slab_v * S

    def ldp(c):
        return plsc.load_gather(
            pbuf, [slab_v, b_v, jnp.full((16,), c, jnp.int32)])

    def ldt(c):
        return plsc.load_gather(
            tbuf, [slab_v, b_v, jnp.full((16,), c, jnp.int32)])

    p = [ldp(c) for c in range(10)]
    t = [ldt(c) for c in range(10)]

    bt = _corners(t[0], t[1], t[2], t[3])
    at_ = _area(bt)
    iou1 = _iou(_corners(p[0], p[1], p[2], p[3]), bt, at_)
    iou2 = _iou(_corners(p[5], p[6], p[7], p[8]), bt, at_)
    sel = iou2 > iou1
    m = jnp.maximum(iou1, iou2)

    vp = [jnp.where(sel, p[5 + k], p[k]) for k in range(5)]
    vt = [jnp.where(sel, t[5 + k], t[k]) for k in range(4)]

    dx = vp[0] - vt[0]
    dy = vp[1] - vt[1]
    xy = dx * dx + dy * dy
    wh = (vp[2] + vt[2] - 2.0 * _sqrt16(vp[2] * vt[2])) + (
        vp[3] + vt[3] - 2.0 * _sqrt16(vp[3] * vt[3])
    )
    dc = vp[4] - m
    conf = dc * dc
    dn0 = p[4] - t[4]
    dn1 = p[9] - t[9]
    no = dn0 * dn0 + dn1 * dn1

    d10 = ldp(10) - ldt(10)
    cls = d10 * d10
    for c in range(11, C):
        d = ldp(c) - ldt(c)
        cls = cls + d * d

    t4 = t[4]
    ow = (t4 != 0.0).astype(jnp.float32)
    nw = (t4 != 1.0).astype(jnp.float32)
    contrib = ow * (
        LAMBDA_LOC * (xy + wh) + conf + 2.0 * cls
    ) + LAMBDA_NOOBJ * nw * no
    contrib = jnp.where(r_raw < CH_ROWS, contrib, 0.0)
    return acc + contrib


def _body(pf_hbm, tf_hbm, out_hbm):
    cid = lax.axis_index("c")
    sid = lax.axis_index("s")
    wid = sid * 2 + cid
    slab_base = wid * SLABS_PER_W

    def inner(*refs):
        pbs = refs[0:RING]
        tbs = refs[RING:2 * RING]
        accv = refs[2 * RING]
        pss = refs[2 * RING + 1:3 * RING + 1]
        tss = refs[3 * RING + 1:4 * RING + 1]

        def start(k, b):
            s0 = slab_base + k * SLABS_CH
            pltpu.async_copy(pf_hbm.at[pl.ds(s0, SLABS_CH)], pbs[b], pss[b])
            pltpu.async_copy(tf_hbm.at[pl.ds(s0, SLABS_CH)], tbs[b], tss[b])

        def wait(b):
            pltpu.make_async_copy(
                pf_hbm.at[pl.ds(0, SLABS_CH)], pbs[b], pss[b]).wait()
            pltpu.make_async_copy(
                tf_hbm.at[pl.ds(0, SLABS_CH)], tbs[b], tss[b]).wait()

        def compute(b, acc):
            return plsc.parallel_loop(0, N_GROUPS, carry=acc)(
                lambda g, a: _group_loss(pbs[b], tbs[b], g, a)
            )

        for j in range(RING - 1):
            start(j, j)

        def quad_body(i, acc):
            k0 = RING * i
            for j in range(RING):
                nk = k0 + j + RING - 1

                @pl.when(nk < N_CHUNKS)
                def _():
                    start(nk, (j + RING - 1) % RING)

                wait(j)
                acc = compute(j, acc)
            return acc

        acc = lax.fori_loop(
            0, N_CHUNKS // RING, quad_body, jnp.zeros((16,), jnp.float32)
        )
        accv[...] = acc
        pltpu.sync_copy(accv, out_hbm.at[wid])

    pl.run_scoped(
        inner,
        *([pltpu.VMEM((SLABS_CH, S, C), jnp.float32)] * (2 * RING)),
        pltpu.VMEM((16,), jnp.float32),
        *([pltpu.SemaphoreType.DMA] * (2 * RING)),
    )


_sc_loss = functools.partial(
    pl.kernel,
    out_type=jax.ShapeDtypeStruct((NW, 16), jnp.float32),
    mesh=plsc.VectorSubcoreMesh(core_axis_name="c", subcore_axis_name="s"),
    compiler_params=pltpu.CompilerParams(
        needs_layout_passes=False, use_tc_tiling_on_sc=True
    ),
)(_body)


@jax.jit
def kernel(predict, target):
    pr = predict.reshape(N_SLABS, S, C)
    tr = target.reshape(N_SLABS, S, C)
    partials = _sc_loss(pr, tr)
    return jnp.sum(partials)


# speedup vs baseline: 1.0497x; 1.0497x over previous
import functools

import jax
import jax.numpy as jnp
from jax import lax
from jax.experimental import pallas as pl
from jax.experimental.pallas import tpu as pltpu
from jax.experimental.pallas import tpu_sc as plsc

S = 7
C = 26
N_IMG = 16384
NW = 32
N_SLABS = N_IMG * S
SLABS_PER_W = N_SLABS // NW
SLABS_CH = 28
N_CHUNKS = SLABS_PER_W // SLABS_CH
CH_ROWS = SLABS_CH * S
N_GROUPS = (CH_ROWS + 15) // 16

LAMBDA_LOC = 10.0
LAMBDA_NOOBJ = 0.5


def _sqrt16(x):
    i = plsc.bitcast(x, jnp.int32)
    r = plsc.bitcast(jnp.int32(0x5F3759DF) - (i >> 1), jnp.float32)
    for _ in range(2):
        r = r * (1.5 - 0.5 * x * r * r)
    return x * r


def _iou_parts(b1, b2, a2):
    xA = jnp.maximum(b1[0], b2[0])
    yA = jnp.maximum(b1[1], b2[1])
    xB = jnp.minimum(b1[2], b2[2])
    yB = jnp.minimum(b1[3], b2[3])
    inter = jnp.maximum(0.0, xB - xA + 1.0) * jnp.maximum(0.0, yB - yA + 1.0)
    a1 = _area(b1)
    return inter, a1 + a2 - inter


def _corners(x, y, w, h):
    x64 = x * 64.0
    y64 = y * 64.0
    w224 = w * 224.0
    h224 = h * 224.0
    return x64 - w224, y64 - h224, x64 + w224, y64 + h224


def _area(b):
    return (b[2] - b[0] + 1.0) * (b[3] - b[1] + 1.0)


def _iou(b1, b2, a2):
    xA = jnp.maximum(b1[0], b2[0])
    yA = jnp.maximum(b1[1], b2[1])
    xB = jnp.minimum(b1[2], b2[2])
    yB = jnp.minimum(b1[3], b2[3])
    inter = jnp.maximum(0.0, xB - xA + 1.0) * jnp.maximum(0.0, yB - yA + 1.0)
    a1 = _area(b1)
    return inter / (a1 + a2 - inter)


def _group_loss(pbuf, tbuf, g, acc):
    lane = lax.iota(jnp.int32, 16)
    r_raw = lane + 16 * g
    r = jnp.minimum(r_raw, CH_ROWS - 1)
    r_f = r.astype(jnp.float32)
    slab_v = (r_f * (1.0 / S) + (0.5 / S)).astype(jnp.int32)
    b_v = r - slab_v * S

    def ldp(c):
        return plsc.load_gather(
            pbuf, [slab_v, b_v, jnp.full((16,), c, jnp.int32)])

    def ldt(c):
        return plsc.load_gather(
            tbuf, [slab_v, b_v, jnp.full((16,), c, jnp.int32)])

    p = [ldp(c) for c in range(10)]
    t = [ldt(c) for c in range(10)]

    bt = _corners(t[0], t[1], t[2], t[3])
    at_ = _area(bt)
    in1, de1 = _iou_parts(_corners(p[0], p[1], p[2], p[3]), bt, at_)
    in2, de2 = _iou_parts(_corners(p[5], p[6], p[7], p[8]), bt, at_)
    sel = in2 * de1 > in1 * de2
    m = jnp.where(sel, in2, in1) / jnp.where(sel, de2, de1)

    vp = [jnp.where(sel, p[5 + k], p[k]) for k in range(5)]
    vt = [jnp.where(sel, t[5 + k], t[k]) for k in range(4)]

    dx = vp[0] - vt[0]
    dy = vp[1] - vt[1]
    xy = dx * dx + dy * dy
    wh = (vp[2] + vt[2] - 2.0 * _sqrt16(vp[2] * vt[2])) + (
        vp[3] + vt[3] - 2.0 * _sqrt16(vp[3] * vt[3])
    )
    dc = vp[4] - m
    conf = dc * dc
    dn0 = p[4] - t[4]
    dn1 = p[9] - t[9]
    no = dn0 * dn0 + dn1 * dn1

    d10 = ldp(10) - ldt(10)
    cls = d10 * d10
    for c in range(11, C):
        d = ldp(c) - ldt(c)
        cls = cls + d * d

    t4 = t[4]
    ow = (t4 != 0.0).astype(jnp.float32)
    nw = (t4 != 1.0).astype(jnp.float32)
    contrib = ow * (
        LAMBDA_LOC * (xy + wh) + conf + 2.0 * cls
    ) + LAMBDA_NOOBJ * nw * no
    contrib = jnp.where(r_raw < CH_ROWS, contrib, 0.0)
    return acc + contrib


def _body(pf_hbm, tf_hbm, out_hbm):
    cid = lax.axis_index("c")
    sid = lax.axis_index("s")
    wid = sid * 2 + cid
    slab_base = wid * SLABS_PER_W

    def inner(pb0, pb1, tb0, tb1, accv, ps0, ps1, ts0, ts1):
        def start(k, pb, tb, psem, tsem):
            s0 = slab_base + k * SLABS_CH
            pltpu.async_copy(pf_hbm.at[pl.ds(s0, SLABS_CH)], pb, psem)
            pltpu.async_copy(tf_hbm.at[pl.ds(s0, SLABS_CH)], tb, tsem)

        def wait(pb, tb, psem, tsem):
            pltpu.make_async_copy(
                pf_hbm.at[pl.ds(0, SLABS_CH)], pb, psem).wait()
            pltpu.make_async_copy(
                tf_hbm.at[pl.ds(0, SLABS_CH)], tb, tsem).wait()

        def compute(pbuf, tbuf, acc):
            return plsc.parallel_loop(0, N_GROUPS, carry=acc)(
                lambda g, a: _group_loss(pbuf, tbuf, g, a)
            )

        start(0, pb0, tb0, ps0, ts0)

        def pair_body(i, acc):
            k0 = 2 * i
            start(k0 + 1, pb1, tb1, ps1, ts1)
            wait(pb0, tb0, ps0, ts0)
            acc = compute(pb0, tb0, acc)

            @pl.when(k0 + 2 < N_CHUNKS)
            def _():
                start(k0 + 2, pb0, tb0, ps0, ts0)

            wait(pb1, tb1, ps1, ts1)
            return compute(pb1, tb1, acc)

        acc = lax.fori_loop(
            0, N_CHUNKS // 2, pair_body, jnp.zeros((16,), jnp.float32)
        )
        accv[...] = acc
        pltpu.sync_copy(accv, out_hbm.at[wid])

    pl.run_scoped(
        inner,
        pltpu.VMEM((SLABS_CH, S, C), jnp.float32),
        pltpu.VMEM((SLABS_CH, S, C), jnp.float32),
        pltpu.VMEM((SLABS_CH, S, C), jnp.float32),
        pltpu.VMEM((SLABS_CH, S, C), jnp.float32),
        pltpu.VMEM((16,), jnp.float32),
        pltpu.SemaphoreType.DMA,
        pltpu.SemaphoreType.DMA,
        pltpu.SemaphoreType.DMA,
        pltpu.SemaphoreType.DMA,
    )


_sc_loss = functools.partial(
    pl.kernel,
    out_type=jax.ShapeDtypeStruct((NW, 16), jnp.float32),
    mesh=plsc.VectorSubcoreMesh(core_axis_name="c", subcore_axis_name="s"),
    compiler_params=pltpu.CompilerParams(
        needs_layout_passes=False, use_tc_tiling_on_sc=True
    ),
)(_body)


@jax.jit
def kernel(predict, target):
    pr = predict.reshape(N_SLABS, S, C)
    tr = target.reshape(N_SLABS, S, C)
    partials = _sc_loss(pr, tr)
    return jnp.sum(partials)


# speedup vs baseline: 1.0667x; 1.0162x over previous
import functools

import jax
import jax.numpy as jnp
from jax import lax
from jax.experimental import pallas as pl
from jax.experimental.pallas import tpu as pltpu
from jax.experimental.pallas import tpu_sc as plsc

S = 7
C = 26
N_IMG = 16384
N_ROWS = N_IMG * S * S
NW = 32
N_SHARDS = 1
CH_ROWS = 784
CH_WORDS = CH_ROWS * C
GROUPS = CH_ROWS // 16

LAMBDA_LOC = 10.0
LAMBDA_NOOBJ = 0.5


def _sqrt16(x):
    i = plsc.bitcast(x, jnp.int32)
    r = plsc.bitcast(jnp.int32(0x5F3759DF) - (i >> 1), jnp.float32)
    for _ in range(2):
        r = r * (1.5 - 0.5 * x * r * r)
    return x * r


def _corners(x, y, w, h):
    x64 = x * 64.0
    y64 = y * 64.0
    w224 = w * 224.0
    h224 = h * 224.0
    return x64 - w224, y64 - h224, x64 + w224, y64 + h224


def _area(b):
    return (b[2] - b[0] + 1.0) * (b[3] - b[1] + 1.0)


def _iou(b1, b2, a2):
    xA = jnp.maximum(b1[0], b2[0])
    yA = jnp.maximum(b1[1], b2[1])
    xB = jnp.minimum(b1[2], b2[2])
    yB = jnp.minimum(b1[3], b2[3])
    inter = jnp.maximum(0.0, xB - xA + 1.0) * jnp.maximum(0.0, yB - yA + 1.0)
    a1 = _area(b1)
    return inter / (a1 + a2 - inter)


def _group_loss(pbuf, tbuf, iota26, g, acc):
    idx0 = iota26 + g * (16 * C)

    def ldp(c):
        return plsc.load_gather(pbuf, [idx0 + c])

    def ldt(c):
        return plsc.load_gather(tbuf, [idx0 + c])

    p = [ldp(c) for c in range(10)]
    t = [ldt(c) for c in range(10)]

    bt = _corners(t[0], t[1], t[2], t[3])
    at_ = _area(bt)
    iou1 = _iou(_corners(p[0], p[1], p[2], p[3]), bt, at_)
    iou2 = _iou(_corners(p[5], p[6], p[7], p[8]), bt, at_)
    sel = iou2 > iou1
    m = jnp.maximum(iou1, iou2)

    vp = [jnp.where(sel, p[5 + k], p[k]) for k in range(5)]
    vt = [jnp.where(sel, t[5 + k], t[k]) for k in range(4)]

    dx = vp[0] - vt[0]
    dy = vp[1] - vt[1]
    xy = dx * dx + dy * dy
    wh = (vp[2] + vt[2] - 2.0 * _sqrt16(vp[2] * vt[2])) + (
        vp[3] + vt[3] - 2.0 * _sqrt16(vp[3] * vt[3])
    )
    dc = vp[4] - m
    conf = dc * dc
    dn0 = p[4] - t[4]
    dn1 = p[9] - t[9]
    no = dn0 * dn0 + dn1 * dn1

    d10 = ldp(10) - ldt(10)
    cls = d10 * d10
    for c in range(11, C):
        d = ldp(c) - ldt(c)
        cls = cls + d * d

    t4 = t[4]
    ow = (t4 != 0.0).astype(jnp.float32)
    nw = (t4 != 1.0).astype(jnp.float32)
    contrib = ow * (
        LAMBDA_LOC * (xy + wh) + conf + 2.0 * cls
    ) + LAMBDA_NOOBJ * nw * no
    return acc + contrib


def _make_body(rows_per_w):
    n_chunks = rows_per_w // CH_ROWS
    assert rows_per_w % CH_ROWS == 0 and n_chunks % 2 == 0

    def _body(pf_hbm, tf_hbm, out_hbm, pb0, pb1, tb0, tb1, accv,
              ps0, ps1, ts0, ts1):
        cid = lax.axis_index("c")
        sid = lax.axis_index("s")
        wid = sid * 2 + cid
        base_elem = wid * (rows_per_w * C)
        iota26 = lax.iota(jnp.int32, 16) * C

        def start(k, pb, tb, psem, tsem):
            off = pl.multiple_of(base_elem + k * CH_WORDS, 8)
            pltpu.async_copy(pf_hbm.at[pl.ds(off, CH_WORDS)], pb, psem)
            pltpu.async_copy(tf_hbm.at[pl.ds(off, CH_WORDS)], tb, tsem)

        def wait(pb, tb, psem, tsem):
            pltpu.make_async_copy(
                pf_hbm.at[pl.ds(0, CH_WORDS)], pb, psem).wait()
            pltpu.make_async_copy(
                tf_hbm.at[pl.ds(0, CH_WORDS)], tb, tsem).wait()

        def compute(pbuf, tbuf, acc):
            return plsc.parallel_loop(0, GROUPS, carry=acc)(
                lambda g, a: _group_loss(pbuf, tbuf, iota26, g, a)
            )

        start(0, pb0, tb0, ps0, ts0)

        def pair_body(i, acc):
            k0 = 2 * i
            start(k0 + 1, pb1, tb1, ps1, ts1)
            wait(pb0, tb0, ps0, ts0)
            acc = compute(pb0, tb0, acc)

            @pl.when(k0 + 2 < n_chunks)
            def _():
                start(k0 + 2, pb0, tb0, ps0, ts0)

            wait(pb1, tb1, ps1, ts1)
            return compute(pb1, tb1, acc)

        acc = lax.fori_loop(
            0, n_chunks // 2, pair_body, jnp.zeros((16,), jnp.float32)
        )
        accv[...] = acc
        pltpu.sync_copy(accv, out_hbm.at[wid])

    return _body


@functools.lru_cache(maxsize=None)
def _make_sc_loss(n_rows):
    return functools.partial(
        pl.kernel,
        out_type=jax.ShapeDtypeStruct((NW, 16), jnp.float32),
        mesh=plsc.VectorSubcoreMesh(core_axis_name="c", subcore_axis_name="s"),
        scratch_types=[
            pltpu.VMEM((CH_WORDS,), jnp.float32),
            pltpu.VMEM((CH_WORDS,), jnp.float32),
            pltpu.VMEM((CH_WORDS,), jnp.float32),
            pltpu.VMEM((CH_WORDS,), jnp.float32),
            pltpu.VMEM((16,), jnp.float32),
            pltpu.SemaphoreType.DMA,
            pltpu.SemaphoreType.DMA,
            pltpu.SemaphoreType.DMA,
            pltpu.SemaphoreType.DMA,
        ],
        compiler_params=pltpu.CompilerParams(needs_layout_passes=False),
    )(_make_body(n_rows // NW))


@jax.jit
def kernel(predict, target):
    imgs_per_shard = N_IMG // N_SHARDS
    rows_per_shard = imgs_per_shard * S * S
    sc_loss = _make_sc_loss(rows_per_shard)
    partials = []
    for k in range(N_SHARDS):
        pf = predict[k * imgs_per_shard:(k + 1) * imgs_per_shard].reshape(-1)
        tf = target[k * imgs_per_shard:(k + 1) * imgs_per_shard].reshape(-1)
        partials.append(sc_loss(pf, tf))
    return jnp.sum(jnp.stack(partials))
